# baseline (speedup 1.0000x reference)
import jax
import jax.numpy as jnp
from jax import lax
from jax.experimental import pallas as pl

GAMMA = 0.99

B = 16384
A = 1024
RB = 2048
NBLK = B // RB


def _body(actual_ref, action_ref, rew_ref, done_ref, out_ref):
    av = action_ref[...]
    ac = actual_ref[...]
    mx = jnp.max(av, axis=1, keepdims=True)
    cols = lax.broadcasted_iota(jnp.int32, (RB, A), 1)
    big = jnp.int32(2**30)
    cstar = jnp.min(jnp.where(av == mx, cols, big), axis=1, keepdims=True)
    mask = cols == cstar
    val = jnp.sum(jnp.where(mask, ac, jnp.float32(0.0)), axis=1, keepdims=True)
    dn = done_ref[...]
    rw = rew_ref[...]
    w = jnp.where(dn != jnp.float32(0.0), jnp.float32(0.0), val)
    out_ref[...] = w * jnp.float32(GAMMA) + rw


def kernel(next_state_actual_values, next_state_action_values, reward, is_done):
    done_f = is_done.astype(jnp.float32)
    out = pl.pallas_call(
        _body,
        grid=(NBLK,),
        in_specs=[
            pl.BlockSpec((RB, A), lambda i: (i, 0)),
            pl.BlockSpec((RB, A), lambda i: (i, 0)),
            pl.BlockSpec((RB, 1), lambda i: (i, 0)),
            pl.BlockSpec((RB, 1), lambda i: (i, 0)),
        ],
        out_specs=pl.BlockSpec((RB, 1), lambda i: (i, 0)),
        out_shape=jax.ShapeDtypeStruct((B, 1), jnp.float32),
    )(next_state_actual_values, next_state_action_values, reward, done_f)
    return out.reshape(B)
